# Initial kernel scaffold; baseline (speedup 1.0000x reference)
import jax
import jax.numpy as jnp
from jax.experimental import pallas as pl


def kernel(x, edge_index, Wl_1, Wr_1, b_1, Wl_1T, Wr_1T, b_1T, Wl_2, Wr_2, b_2, Wl_2T, Wr_2T, b_2T):
    raise NotImplementedError("write your pallas kernel here")


# speedup vs baseline: 5.5252x; 5.5252x over previous
import functools

import jax
import jax.numpy as jnp
from jax import lax
from jax.experimental import pallas as pl
from jax.experimental.pallas import tpu as pltpu
from jax.experimental.pallas import tpu_sc as plsc

N = 10000
E = 320000
D = 128
DP = 144
NC, NS = 2, 16
RPT = N // NS
CW = 80
CHUNKS = E // (NS * CW)
G = 50
NG = CHUNKS // G
ZR = 25


def _segsum_body(table, eidx, out, acc, gidx, sidx, rows, zbuf, sem):
    c = lax.axis_index("c")
    s = lax.axis_index("s")

    def zstore(t, _):
        r = t // (DP // 16)
        j = t % (DP // 16)
        zbuf[r, pl.ds(j * 16, 16)] = jnp.zeros((16,), jnp.float32)
        return 0
    lax.fori_loop(0, ZR * (DP // 16), zstore, 0)

    def zcopy(k, _):
        pltpu.sync_copy(zbuf.at[pl.ds(0, ZR)],
                        acc.at[pl.ds(s * RPT + k * ZR, ZR)])
        return 0
    lax.fori_loop(0, RPT // ZR, zcopy, 0)

    plsc.subcore_barrier()

    def group(g, _):
        pltpu.sync_copy(eidx.at[c, pl.ds(s * CHUNKS + g * G, G)], gidx)
        pltpu.sync_copy(eidx.at[1 - c, pl.ds(s * CHUNKS + g * G, G)], sidx)

        def step(i, _):
            pltpu.async_copy(table.at[gidx.at[i]], rows, sem).wait()
            pltpu.sync_copy(rows, acc.at[sidx.at[i]], add=True)
            return 0
        lax.fori_loop(0, G, step, 0)
        return 0
    lax.fori_loop(0, NG, group, 0)

    plsc.subcore_barrier()

    pltpu.sync_copy(acc.at[pl.ds(s * RPT, RPT)],
                    out.at[c, pl.ds(s * RPT, RPT)])


@jax.jit
def _segsum(table, eidx):
    mesh = plsc.VectorSubcoreMesh(core_axis_name="c", subcore_axis_name="s",
                                  num_cores=NC, num_subcores=NS)
    f = pl.kernel(
        _segsum_body,
        out_type=jax.ShapeDtypeStruct((2, N, DP), jnp.float32),
        mesh=mesh,
        scratch_types=[
            pltpu.VMEM_SHARED((N, DP), jnp.float32),
            pltpu.VMEM((G, CW), jnp.int32),
            pltpu.VMEM((G, CW), jnp.int32),
            pltpu.VMEM((CW, DP), jnp.float32),
            pltpu.VMEM((ZR, DP), jnp.float32),
            pltpu.SemaphoreType.DMA,
        ],
        compiler_params=pltpu.CompilerParams(use_tc_tiling_on_sc=False),
    )
    return f(table, eidx)


def _dense_body(sums_f, sums_b, xin, wl, wlt, wr, wrt, b, bt, out, *, pad):
    sf = sums_f[0]
    sb = sums_b[0]
    mf = sf[:, :D] / jnp.maximum(sf[:, D:D + 1], 1.0)
    mb = sb[:, :D] / jnp.maximum(sb[:, D:D + 1], 1.0)
    xs = xin[...][:, :D]
    w_self = wr[...] + wrt[...]
    h = (jnp.dot(mf, wl[...], preferred_element_type=jnp.float32)
         + jnp.dot(mb, wlt[...], preferred_element_type=jnp.float32)
         + jnp.dot(xs, w_self, preferred_element_type=jnp.float32)
         + b[...] + bt[...])
    if pad:
        out[:, :D] = h
        out[:, D:D + 1] = jnp.ones_like(out[:, D:D + 1])
        out[:, D + 1:] = jnp.zeros_like(out[:, D + 1:])
    else:
        out[...] = h


@functools.partial(jax.jit, static_argnames=("pad",))
def _dense(sums, xin, wl, wlt, wr, wrt, b, bt, *, pad):
    blk = 1000
    d_out = DP if pad else D
    grid = (N // blk,)
    return pl.pallas_call(
        functools.partial(_dense_body, pad=pad),
        grid=grid,
        in_specs=[
            pl.BlockSpec((1, blk, DP), lambda i: (0, i, 0)),
            pl.BlockSpec((1, blk, DP), lambda i: (1, i, 0)),
            pl.BlockSpec((blk, DP), lambda i: (i, 0)),
            pl.BlockSpec((D, D), lambda i: (0, 0)),
            pl.BlockSpec((D, D), lambda i: (0, 0)),
            pl.BlockSpec((D, D), lambda i: (0, 0)),
            pl.BlockSpec((D, D), lambda i: (0, 0)),
            pl.BlockSpec((1, D), lambda i: (0, 0)),
            pl.BlockSpec((1, D), lambda i: (0, 0)),
        ],
        out_specs=pl.BlockSpec((blk, d_out), lambda i: (i, 0)),
        out_shape=jax.ShapeDtypeStruct((N, d_out), jnp.float32),
    )(sums, sums, xin, wl, wlt, wr, wrt, b, bt)


def kernel(x, edge_index, Wl_1, Wr_1, b_1, Wl_1T, Wr_1T, b_1T,
           Wl_2, Wr_2, b_2, Wl_2T, Wr_2T, b_2T):
    ei = edge_index.astype(jnp.int32).reshape(2, E // CW, CW)
    x_aug = jnp.concatenate(
        [x, jnp.ones((N, 1), jnp.float32), jnp.zeros((N, DP - D - 1), jnp.float32)],
        axis=1)
    b_1 = b_1.reshape(1, D)
    b_1T = b_1T.reshape(1, D)
    b_2 = b_2.reshape(1, D)
    b_2T = b_2T.reshape(1, D)

    sums1 = _segsum(x_aug, ei)
    h_aug = _dense(sums1, x_aug, Wl_1, Wl_1T, Wr_1, Wr_1T, b_1, b_1T, pad=True)
    sums2 = _segsum(h_aug, ei)
    out = _dense(sums2, h_aug, Wl_2, Wl_2T, Wr_2, Wr_2T, b_2, b_2T, pad=False)
    return out


# speedup vs baseline: 8.6523x; 1.5660x over previous
import functools

import jax
import jax.numpy as jnp
from jax import lax
from jax.experimental import pallas as pl
from jax.experimental.pallas import tpu as pltpu
from jax.experimental.pallas import tpu_sc as plsc

N = 10000
E = 320000
D = 128
DP = 144
NC, NS = 2, 16
RPT = N // NS
CW = 80
CHUNKS = E // (NS * CW)
G = 50
NG = CHUNKS // G
NB = 2


def _segsum_body(table, eidx, out, acc, gidx, sidx, rows, sem_g, sem_s):
    c = lax.axis_index("c")
    s = lax.axis_index("s")
    base = s * CHUNKS

    def zstore(t, _):
        r = t // (DP // 16)
        j = t % (DP // 16)
        rows[0, r, pl.ds(j * 16, 16)] = jnp.zeros((16,), jnp.float32)
        return 0
    lax.fori_loop(0, CW * (DP // 16), zstore, 0)
    for k in range(RPT // CW):
        pltpu.sync_copy(rows.at[0, pl.ds(0, CW)],
                        acc.at[pl.ds(s * RPT + k * CW, CW)])
    pltpu.sync_copy(rows.at[0, pl.ds(0, RPT % CW)],
                    acc.at[pl.ds(s * RPT + (RPT // CW) * CW, RPT % CW)])

    pltpu.sync_copy(eidx.at[c, pl.ds(base, G)], gidx.at[0])
    pltpu.sync_copy(eidx.at[1 - c, pl.ds(base, G)], sidx.at[0])
    for j in range(NB):
        pltpu.async_copy(table.at[gidx.at[0, j]], rows.at[j], sem_g)

    plsc.subcore_barrier()

    def step(i, _):
        grp = i // G
        pos = i % G
        slot = grp % 2
        buf = i % NB

        @pl.when(jnp.logical_and(pos == 0, grp + 1 < NG))
        def _():
            nbase = base + (grp + 1) * G
            pltpu.sync_copy(eidx.at[c, pl.ds(nbase, G)], gidx.at[(grp + 1) % 2])
            pltpu.sync_copy(eidx.at[1 - c, pl.ds(nbase, G)], sidx.at[(grp + 1) % 2])

        pltpu.make_async_copy(table.at[gidx.at[slot, pos]], rows.at[buf],
                              sem_g).wait()
        pltpu.sync_copy(rows.at[buf], acc.at[sidx.at[slot, pos]], add=True)

        nxt = i + NB

        @pl.when(nxt < CHUNKS)
        def _():
            pltpu.async_copy(table.at[gidx.at[(nxt // G) % 2, nxt % G]],
                             rows.at[buf], sem_g)
        return 0
    lax.fori_loop(0, CHUNKS, step, 0)

    plsc.subcore_barrier()

    pltpu.sync_copy(acc.at[pl.ds(s * RPT, RPT)],
                    out.at[c, pl.ds(s * RPT, RPT)])


@jax.jit
def _segsum(table, eidx):
    mesh = plsc.VectorSubcoreMesh(core_axis_name="c", subcore_axis_name="s",
                                  num_cores=NC, num_subcores=NS)
    f = pl.kernel(
        _segsum_body,
        out_type=jax.ShapeDtypeStruct((2, N, DP), jnp.float32),
        mesh=mesh,
        scratch_types=[
            pltpu.VMEM_SHARED((N, DP), jnp.float32),
            pltpu.VMEM((2, G, CW), jnp.int32),
            pltpu.VMEM((2, G, CW), jnp.int32),
            pltpu.VMEM((NB, CW, DP), jnp.float32),
            pltpu.SemaphoreType.DMA,
            pltpu.SemaphoreType.DMA,
        ],
        compiler_params=pltpu.CompilerParams(use_tc_tiling_on_sc=False),
    )
    return f(table, eidx)


def _dense_body(sums_f, sums_b, xin, wl, wlt, wr, wrt, b, bt, out, *, pad):
    sf = sums_f[0]
    sb = sums_b[0]
    mf = sf[:, :D] / jnp.maximum(sf[:, D:D + 1], 1.0)
    mb = sb[:, :D] / jnp.maximum(sb[:, D:D + 1], 1.0)
    xs = xin[...][:, :D]
    w_self = wr[...] + wrt[...]
    h = (jnp.dot(mf, wl[...], preferred_element_type=jnp.float32)
         + jnp.dot(mb, wlt[...], preferred_element_type=jnp.float32)
         + jnp.dot(xs, w_self, preferred_element_type=jnp.float32)
         + b[...] + bt[...])
    if pad:
        out[:, :D] = h
        out[:, D:D + 1] = jnp.ones_like(out[:, D:D + 1])
        out[:, D + 1:] = jnp.zeros_like(out[:, D + 1:])
    else:
        out[...] = h


@functools.partial(jax.jit, static_argnames=("pad",))
def _dense(sums, xin, wl, wlt, wr, wrt, b, bt, *, pad):
    blk = 1000
    d_out = DP if pad else D
    grid = (N // blk,)
    return pl.pallas_call(
        functools.partial(_dense_body, pad=pad),
        grid=grid,
        in_specs=[
            pl.BlockSpec((1, blk, DP), lambda i: (0, i, 0)),
            pl.BlockSpec((1, blk, DP), lambda i: (1, i, 0)),
            pl.BlockSpec((blk, DP), lambda i: (i, 0)),
            pl.BlockSpec((D, D), lambda i: (0, 0)),
            pl.BlockSpec((D, D), lambda i: (0, 0)),
            pl.BlockSpec((D, D), lambda i: (0, 0)),
            pl.BlockSpec((D, D), lambda i: (0, 0)),
            pl.BlockSpec((1, D), lambda i: (0, 0)),
            pl.BlockSpec((1, D), lambda i: (0, 0)),
        ],
        out_specs=pl.BlockSpec((blk, d_out), lambda i: (i, 0)),
        out_shape=jax.ShapeDtypeStruct((N, d_out), jnp.float32),
    )(sums, sums, xin, wl, wlt, wr, wrt, b, bt)


def kernel(x, edge_index, Wl_1, Wr_1, b_1, Wl_1T, Wr_1T, b_1T,
           Wl_2, Wr_2, b_2, Wl_2T, Wr_2T, b_2T):
    ei = edge_index.astype(jnp.int32).reshape(2, E // CW, CW)
    x_aug = jnp.concatenate(
        [x, jnp.ones((N, 1), jnp.float32), jnp.zeros((N, DP - D - 1), jnp.float32)],
        axis=1)
    b_1 = b_1.reshape(1, D)
    b_1T = b_1T.reshape(1, D)
    b_2 = b_2.reshape(1, D)
    b_2T = b_2T.reshape(1, D)

    sums1 = _segsum(x_aug, ei)
    h_aug = _dense(sums1, x_aug, Wl_1, Wl_1T, Wr_1, Wr_1T, b_1, b_1T, pad=True)
    sums2 = _segsum(h_aug, ei)
    out = _dense(sums2, h_aug, Wl_2, Wl_2T, Wr_2, Wr_2T, b_2, b_2T, pad=False)
    return out


# speedup vs baseline: 9.2984x; 1.0747x over previous
import functools

import jax
import jax.numpy as jnp
from jax import lax
from jax.experimental import pallas as pl
from jax.experimental.pallas import tpu as pltpu
from jax.experimental.pallas import tpu_sc as plsc

N = 10000
E = 320000
D = 128
DP = 144
NC, NS = 2, 16
RPT = N // NS
CW = 80
CHUNKS = E // (NS * CW)
G = 10
NG = CHUNKS // G
NB = 3


def _segsum_body(table, eidx, out, acc, gidx, sidx, rows, sem_g, sem_s):
    c = lax.axis_index("c")
    s = lax.axis_index("s")
    base = s * CHUNKS

    def zstore(t, _):
        r = t // (DP // 16)
        j = t % (DP // 16)
        rows[0, r, pl.ds(j * 16, 16)] = jnp.zeros((16,), jnp.float32)
        return 0
    lax.fori_loop(0, CW * (DP // 16), zstore, 0)
    for k in range(RPT // CW):
        pltpu.sync_copy(rows.at[0, pl.ds(0, CW)],
                        acc.at[pl.ds(s * RPT + k * CW, CW)])
    pltpu.sync_copy(rows.at[0, pl.ds(0, RPT % CW)],
                    acc.at[pl.ds(s * RPT + (RPT // CW) * CW, RPT % CW)])

    pltpu.sync_copy(eidx.at[c, pl.ds(base, G)], gidx.at[0])
    pltpu.sync_copy(eidx.at[1 - c, pl.ds(base, G)], sidx.at[0])
    for j in range(NB):
        pltpu.async_copy(table.at[gidx.at[0, j]], rows.at[j], sem_g)

    plsc.subcore_barrier()

    def step(i, _):
        grp = i // G
        pos = i % G
        slot = grp % 2
        buf = i % NB

        pltpu.make_async_copy(table.at[gidx.at[slot, pos]], rows.at[buf],
                              sem_g).wait()

        @pl.when(i < CHUNKS - 1)
        def _():
            pltpu.async_copy(rows.at[buf], acc.at[sidx.at[slot, pos]], sem_s,
                             add=True)

        @pl.when(i == CHUNKS - 1)
        def _():
            pltpu.sync_copy(rows.at[buf], acc.at[sidx.at[slot, pos]], add=True)

        @pl.when(i > 0)
        def _():
            pltpu.make_async_copy(
                rows.at[(i - 1) % NB],
                acc.at[sidx.at[((i - 1) // G) % 2, (i - 1) % G]], sem_s).wait()

        @pl.when(jnp.logical_and(pos == 0, grp + 1 < NG))
        def _():
            nbase = base + (grp + 1) * G
            pltpu.sync_copy(eidx.at[c, pl.ds(nbase, G)], gidx.at[(grp + 1) % 2])
            pltpu.sync_copy(eidx.at[1 - c, pl.ds(nbase, G)], sidx.at[(grp + 1) % 2])

        nxt = i + NB - 1

        @pl.when(jnp.logical_and(i > 0, nxt < CHUNKS))
        def _():
            pltpu.async_copy(table.at[gidx.at[(nxt // G) % 2, nxt % G]],
                             rows.at[(i - 1) % NB], sem_g)
        return 0
    lax.fori_loop(0, CHUNKS, step, 0)

    plsc.subcore_barrier()

    pltpu.sync_copy(acc.at[pl.ds(s * RPT, RPT)],
                    out.at[c, pl.ds(s * RPT, RPT)])


@jax.jit
def _segsum(table, eidx):
    mesh = plsc.VectorSubcoreMesh(core_axis_name="c", subcore_axis_name="s",
                                  num_cores=NC, num_subcores=NS)
    f = pl.kernel(
        _segsum_body,
        out_type=jax.ShapeDtypeStruct((2, N, DP), jnp.float32),
        mesh=mesh,
        scratch_types=[
            pltpu.VMEM_SHARED((N, DP), jnp.float32),
            pltpu.VMEM((2, G, CW), jnp.int32),
            pltpu.VMEM((2, G, CW), jnp.int32),
            pltpu.VMEM((NB, CW, DP), jnp.float32),
            pltpu.SemaphoreType.DMA,
            pltpu.SemaphoreType.DMA,
        ],
        compiler_params=pltpu.CompilerParams(use_tc_tiling_on_sc=False),
    )
    return f(table, eidx)


def _dense_body(sums_f, sums_b, xin, wl, wlt, wr, wrt, b, bt, out, *, pad):
    sf = sums_f[0]
    sb = sums_b[0]
    mf = sf[:, :D] / jnp.maximum(sf[:, D:D + 1], 1.0)
    mb = sb[:, :D] / jnp.maximum(sb[:, D:D + 1], 1.0)
    xs = xin[...][:, :D]
    w_self = wr[...] + wrt[...]
    h = (jnp.dot(mf, wl[...], preferred_element_type=jnp.float32)
         + jnp.dot(mb, wlt[...], preferred_element_type=jnp.float32)
         + jnp.dot(xs, w_self, preferred_element_type=jnp.float32)
         + b[...] + bt[...])
    if pad:
        out[:, :D] = h
        out[:, D:D + 1] = jnp.ones_like(out[:, D:D + 1])
        out[:, D + 1:] = jnp.zeros_like(out[:, D + 1:])
    else:
        out[...] = h


@functools.partial(jax.jit, static_argnames=("pad",))
def _dense(sums, xin, wl, wlt, wr, wrt, b, bt, *, pad):
    blk = 1000
    d_out = DP if pad else D
    grid = (N // blk,)
    return pl.pallas_call(
        functools.partial(_dense_body, pad=pad),
        grid=grid,
        in_specs=[
            pl.BlockSpec((1, blk, DP), lambda i: (0, i, 0)),
            pl.BlockSpec((1, blk, DP), lambda i: (1, i, 0)),
            pl.BlockSpec((blk, DP), lambda i: (i, 0)),
            pl.BlockSpec((D, D), lambda i: (0, 0)),
            pl.BlockSpec((D, D), lambda i: (0, 0)),
            pl.BlockSpec((D, D), lambda i: (0, 0)),
            pl.BlockSpec((D, D), lambda i: (0, 0)),
            pl.BlockSpec((1, D), lambda i: (0, 0)),
            pl.BlockSpec((1, D), lambda i: (0, 0)),
        ],
        out_specs=pl.BlockSpec((blk, d_out), lambda i: (i, 0)),
        out_shape=jax.ShapeDtypeStruct((N, d_out), jnp.float32),
    )(sums, sums, xin, wl, wlt, wr, wrt, b, bt)


def kernel(x, edge_index, Wl_1, Wr_1, b_1, Wl_1T, Wr_1T, b_1T,
           Wl_2, Wr_2, b_2, Wl_2T, Wr_2T, b_2T):
    ei = edge_index.astype(jnp.int32).reshape(2, E // CW, CW)
    x_aug = jnp.concatenate(
        [x, jnp.ones((N, 1), jnp.float32), jnp.zeros((N, DP - D - 1), jnp.float32)],
        axis=1)
    b_1 = b_1.reshape(1, D)
    b_1T = b_1T.reshape(1, D)
    b_2 = b_2.reshape(1, D)
    b_2T = b_2T.reshape(1, D)

    sums1 = _segsum(x_aug, ei)
    h_aug = _dense(sums1, x_aug, Wl_1, Wl_1T, Wr_1, Wr_1T, b_1, b_1T, pad=True)
    sums2 = _segsum(h_aug, ei)
    out = _dense(sums2, h_aug, Wl_2, Wl_2T, Wr_2, Wr_2T, b_2, b_2T, pad=False)
    return out


# speedup vs baseline: 10.5967x; 1.1396x over previous
import functools

import jax
import jax.numpy as jnp
from jax import lax
from jax.experimental import pallas as pl
from jax.experimental.pallas import tpu as pltpu
from jax.experimental.pallas import tpu_sc as plsc

N = 10000
E = 320000
D = 128
DP = 144
NC, NS = 2, 16
RPT = N // NS
CW = 80
CHUNKS = E // (NS * CW)
G = 10
NG = CHUNKS // G


def _segsum_body(table, eidx, out, acc, gidx, sidx, rows, sem_g, sem_s,
                 *, dp, nb):
    c = lax.axis_index("c")
    s = lax.axis_index("s")
    base = s * CHUNKS

    def zstore(t, _):
        r = t // (dp // 16)
        j = t % (dp // 16)
        rows[0, r, pl.ds(j * 16, 16)] = jnp.zeros((16,), jnp.float32)
        return 0
    lax.fori_loop(0, CW * (dp // 16), zstore, 0)
    for k in range(RPT // CW):
        pltpu.sync_copy(rows.at[0, pl.ds(0, CW)],
                        acc.at[pl.ds(s * RPT + k * CW, CW)])
    pltpu.sync_copy(rows.at[0, pl.ds(0, RPT % CW)],
                    acc.at[pl.ds(s * RPT + (RPT // CW) * CW, RPT % CW)])

    pltpu.sync_copy(eidx.at[c, pl.ds(base, G)], gidx.at[0])
    pltpu.sync_copy(eidx.at[1 - c, pl.ds(base, G)], sidx.at[0])
    for j in range(nb):
        pltpu.async_copy(table.at[gidx.at[0, j]], rows.at[j], sem_g)

    plsc.subcore_barrier()

    def step(i, _):
        grp = i // G
        pos = i % G
        slot = grp % 2
        buf = i % nb

        pltpu.make_async_copy(table.at[gidx.at[slot, pos]], rows.at[buf],
                              sem_g).wait()

        @pl.when(i < CHUNKS - 1)
        def _():
            pltpu.async_copy(rows.at[buf], acc.at[sidx.at[slot, pos]], sem_s,
                             add=True)

        @pl.when(i == CHUNKS - 1)
        def _():
            pltpu.sync_copy(rows.at[buf], acc.at[sidx.at[slot, pos]], add=True)

        @pl.when(i > 0)
        def _():
            pltpu.make_async_copy(
                rows.at[(i - 1) % nb],
                acc.at[sidx.at[((i - 1) // G) % 2, (i - 1) % G]], sem_s).wait()

        @pl.when(jnp.logical_and(pos == 0, grp + 1 < NG))
        def _():
            nbase = base + (grp + 1) * G
            pltpu.sync_copy(eidx.at[c, pl.ds(nbase, G)], gidx.at[(grp + 1) % 2])
            pltpu.sync_copy(eidx.at[1 - c, pl.ds(nbase, G)], sidx.at[(grp + 1) % 2])

        nxt = i + nb - 1

        @pl.when(jnp.logical_and(i > 0, nxt < CHUNKS))
        def _():
            pltpu.async_copy(table.at[gidx.at[(nxt // G) % 2, nxt % G]],
                             rows.at[(i - 1) % nb], sem_g)
        return 0
    lax.fori_loop(0, CHUNKS, step, 0)

    plsc.subcore_barrier()

    pltpu.sync_copy(acc.at[pl.ds(s * RPT, RPT)],
                    out.at[c, pl.ds(s * RPT, RPT)])


@functools.partial(jax.jit, static_argnames=("dp", "nb"))
def _segsum(table, eidx, *, dp, nb):
    mesh = plsc.VectorSubcoreMesh(core_axis_name="c", subcore_axis_name="s",
                                  num_cores=NC, num_subcores=NS)
    f = pl.kernel(
        functools.partial(_segsum_body, dp=dp, nb=nb),
        out_type=jax.ShapeDtypeStruct((2, N, dp), jnp.float32),
        mesh=mesh,
        scratch_types=[
            pltpu.VMEM_SHARED((N, dp), jnp.float32),
            pltpu.VMEM((2, G, CW), jnp.int32),
            pltpu.VMEM((2, G, CW), jnp.int32),
            pltpu.VMEM((nb, CW, dp), jnp.float32),
            pltpu.SemaphoreType.DMA,
            pltpu.SemaphoreType.DMA,
        ],
        compiler_params=pltpu.CompilerParams(use_tc_tiling_on_sc=False),
    )
    return f(table, eidx)


def _dense_body(sums_f, sums_b, cnt_f, cnt_b, xin, wl, wlt, wr, wrt, b, bt,
                out, *, dp_s):
    sf = sums_f[0][:, :D]
    sb = sums_b[0][:, :D]
    mf = sf / jnp.maximum(cnt_f[0][:, D:D + 1], 1.0)
    mb = sb / jnp.maximum(cnt_b[0][:, D:D + 1], 1.0)
    xs = xin[...]
    w_self = wr[...] + wrt[...]
    out[...] = (jnp.dot(mf, wl[...], preferred_element_type=jnp.float32)
                + jnp.dot(mb, wlt[...], preferred_element_type=jnp.float32)
                + jnp.dot(xs, w_self, preferred_element_type=jnp.float32)
                + b[...] + bt[...])


@functools.partial(jax.jit, static_argnames=("dp_s",))
def _dense(sums, cnts, xin, wl, wlt, wr, wrt, b, bt, *, dp_s):
    blk = 1000
    grid = (N // blk,)
    return pl.pallas_call(
        functools.partial(_dense_body, dp_s=dp_s),
        grid=grid,
        in_specs=[
            pl.BlockSpec((1, blk, dp_s), lambda i: (0, i, 0)),
            pl.BlockSpec((1, blk, dp_s), lambda i: (1, i, 0)),
            pl.BlockSpec((1, blk, DP), lambda i: (0, i, 0)),
            pl.BlockSpec((1, blk, DP), lambda i: (1, i, 0)),
            pl.BlockSpec((blk, D), lambda i: (i, 0)),
            pl.BlockSpec((D, D), lambda i: (0, 0)),
            pl.BlockSpec((D, D), lambda i: (0, 0)),
            pl.BlockSpec((D, D), lambda i: (0, 0)),
            pl.BlockSpec((D, D), lambda i: (0, 0)),
            pl.BlockSpec((1, D), lambda i: (0, 0)),
            pl.BlockSpec((1, D), lambda i: (0, 0)),
        ],
        out_specs=pl.BlockSpec((blk, D), lambda i: (i, 0)),
        out_shape=jax.ShapeDtypeStruct((N, D), jnp.float32),
    )(sums, sums, cnts, cnts, xin, wl, wlt, wr, wrt, b, bt)


def kernel(x, edge_index, Wl_1, Wr_1, b_1, Wl_1T, Wr_1T, b_1T,
           Wl_2, Wr_2, b_2, Wl_2T, Wr_2T, b_2T):
    ei = edge_index.astype(jnp.int32).reshape(2, E // CW, CW)
    x_aug = jnp.concatenate(
        [x, jnp.ones((N, 1), jnp.float32),
         jnp.zeros((N, DP - D - 1), jnp.float32)], axis=1)
    b_1 = b_1.reshape(1, D)
    b_1T = b_1T.reshape(1, D)
    b_2 = b_2.reshape(1, D)
    b_2T = b_2T.reshape(1, D)

    sums1 = _segsum(x_aug, ei, dp=DP, nb=3)
    h = _dense(sums1, sums1, x, Wl_1, Wl_1T, Wr_1, Wr_1T, b_1, b_1T, dp_s=DP)
    sums2 = _segsum(h, ei, dp=D, nb=4)
    out = _dense(sums2, sums1, h, Wl_2, Wl_2T, Wr_2, Wr_2T, b_2, b_2T, dp_s=D)
    return out
